# Initial kernel scaffold; baseline (speedup 1.0000x reference)
import jax
import jax.numpy as jnp
from jax import lax
from jax.experimental import pallas as pl
from jax.experimental.pallas import tpu as pltpu
from jax.experimental.pallas import tpu_sc as plsc

B = 16384
F = 26
K = 16
NC = 2
NS = 16
NW = NC * NS
BPW = B // NW
GB = 4
IPS = GB * F
STEPS = BPW // GB


def _fm_body(idx_hbm, table_hbm, out_hbm, idx_v, buf0, buf1, out_v, sem0, sem1):
    wid = lax.axis_index("s") * NC + lax.axis_index("c")

    pltpu.sync_copy(idx_hbm.at[wid], idx_v)

    def start(j, buf, sem):
        pltpu.async_copy(table_hbm.at[idx_v.at[j]], buf, sem)

    def wait(buf, sem):
        pltpu.make_async_copy(table_hbm.at[idx_v.at[0]], buf, sem).wait()

    def compute(buf, out_base):
        for be in range(GB):
            v = buf[be * F]
            s = v
            q = v * v
            for f in range(1, F):
                v = buf[be * F + f]
                s = s + v
                q = q + v * v
            r = s * s - q
            out_v[out_base + be] = 0.5 * jnp.sum(r)

    start(0, buf0, sem0)
    start(1, buf1, sem1)

    @pl.loop(0, STEPS - 2, step=2)
    def _(g):
        wait(buf0, sem0)
        compute(buf0, g * GB)
        start(g + 2, buf0, sem0)
        wait(buf1, sem1)
        compute(buf1, (g + 1) * GB)
        start(g + 3, buf1, sem1)

    wait(buf0, sem0)
    compute(buf0, (STEPS - 2) * GB)
    wait(buf1, sem1)
    compute(buf1, (STEPS - 1) * GB)

    pltpu.sync_copy(out_v, out_hbm.at[pl.ds(wid * BPW, BPW)])


@jax.jit
def kernel(indices, player_v):
    idx3 = indices.astype(jnp.int32).reshape(NW, STEPS, IPS)
    mesh = plsc.VectorSubcoreMesh(
        core_axis_name="c", subcore_axis_name="s", num_cores=NC, num_subcores=NS
    )
    fm = pl.kernel(
        _fm_body,
        out_type=jax.ShapeDtypeStruct((B,), jnp.float32),
        mesh=mesh,
        scratch_types=[
            pltpu.VMEM((STEPS, IPS), jnp.int32),
            pltpu.VMEM((IPS, K), jnp.float32),
            pltpu.VMEM((IPS, K), jnp.float32),
            pltpu.VMEM((BPW,), jnp.float32),
            pltpu.SemaphoreType.DMA,
            pltpu.SemaphoreType.DMA,
        ],
    )
    return fm(idx3, player_v)


# speedup vs baseline: 1.2438x; 1.2438x over previous
import jax
import jax.numpy as jnp
from jax import lax
from jax.experimental import pallas as pl
from jax.experimental.pallas import tpu as pltpu
from jax.experimental.pallas import tpu_sc as plsc

B = 16384
F = 26
K = 16
NC = 2
NS = 16
NW = NC * NS
BPW = B // NW
GB = 4
IPS = GB * F
GPC = 4
CB = GB * GPC
CHUNKS = BPW // CB
NGATHER = BPW // GB


def _fm_body(idx_hbm, table_hbm, out_hbm, idx_v, buf0, buf1, out_v, sem0, sem1):
    wid = lax.axis_index("s") * NC + lax.axis_index("c")

    pltpu.sync_copy(idx_hbm.at[wid], idx_v)

    lane = lax.iota(jnp.int32, 16)

    def start(c, buf, sem):
        for u in range(GPC):
            pltpu.async_copy(
                table_hbm.at[idx_v.at[c * GPC + u]],
                buf.at[pl.ds(u * IPS, IPS)],
                sem,
            )

    def wait(buf, sem):
        for u in range(GPC):
            pltpu.make_async_copy(
                table_hbm.at[idx_v.at[0]], buf.at[pl.ds(u * IPS, IPS)], sem
            ).wait()

    def compute(buf, c):
        acc = jnp.zeros((16,), jnp.float32)
        for be in range(CB):
            v = buf[be * F]
            s = v
            q = v * v
            for f in range(1, F):
                v = buf[be * F + f]
                s = s + v
                q = q + v * v
            r = s * s - q
            acc = jnp.where(lane == be, jnp.sum(r), acc)
        out_v[pl.ds(c * CB, CB)] = acc * 0.5

    start(0, buf0, sem0)
    start(1, buf1, sem1)

    @pl.loop(0, CHUNKS - 2, step=2)
    def _(g):
        wait(buf0, sem0)
        compute(buf0, g)
        start(g + 2, buf0, sem0)
        wait(buf1, sem1)
        compute(buf1, g + 1)
        start(g + 3, buf1, sem1)

    wait(buf0, sem0)
    compute(buf0, CHUNKS - 2)
    wait(buf1, sem1)
    compute(buf1, CHUNKS - 1)

    pltpu.sync_copy(out_v, out_hbm.at[pl.ds(wid * BPW, BPW)])


@jax.jit
def kernel(indices, player_v):
    idx3 = indices.astype(jnp.int32).reshape(NW, NGATHER, IPS)
    mesh = plsc.VectorSubcoreMesh(
        core_axis_name="c", subcore_axis_name="s", num_cores=NC, num_subcores=NS
    )
    fm = pl.kernel(
        _fm_body,
        out_type=jax.ShapeDtypeStruct((B,), jnp.float32),
        mesh=mesh,
        compiler_params=pltpu.CompilerParams(
            needs_layout_passes=False, use_tc_tiling_on_sc=False
        ),
        scratch_types=[
            pltpu.VMEM((NGATHER, IPS), jnp.int32),
            pltpu.VMEM((CB * F, K), jnp.float32),
            pltpu.VMEM((CB * F, K), jnp.float32),
            pltpu.VMEM((BPW,), jnp.float32),
            pltpu.SemaphoreType.DMA,
            pltpu.SemaphoreType.DMA,
        ],
    )
    return fm(idx3, player_v)


# speedup vs baseline: 1.2618x; 1.0145x over previous
import jax
import jax.numpy as jnp
from jax import lax
from jax.experimental import pallas as pl
from jax.experimental.pallas import tpu as pltpu
from jax.experimental.pallas import tpu_sc as plsc

B = 16384
F = 26
K = 16
NC = 2
NS = 16
NW = NC * NS
BPW = B // NW
CB = 128
CHUNKS = BPW // CB
SB = 16
NBLK = CB // SB


def _fm_body(idx_hbm, table_hbm, out_hbm, idx_v, buf0, buf1, out_v, isem, sem0, sem1):
    wid = lax.axis_index("s") * NC + lax.axis_index("c")
    wbase = wid * BPW

    for f in range(F):
        pltpu.async_copy(idx_hbm.at[f, pl.ds(wbase, BPW)], idx_v.at[f], isem)
    for f in range(F):
        pltpu.make_async_copy(
            idx_hbm.at[0, pl.ds(wbase, BPW)], idx_v.at[f], isem
        ).wait()

    lane = lax.iota(jnp.int32, 16)

    def start(c, buf, sem):
        for f in range(F):
            pltpu.async_copy(
                table_hbm.at[idx_v.at[f, pl.ds(c * CB, CB)]],
                buf.at[pl.ds(f * CB, CB)],
                sem,
            )

    def wait(buf, sem):
        for f in range(F):
            pltpu.make_async_copy(
                table_hbm.at[idx_v.at[0, pl.ds(0, CB)]],
                buf.at[pl.ds(f * CB, CB)],
                sem,
            ).wait()

    def compute(buf, c):
        @pl.loop(0, NBLK)
        def _(sb):
            base = sb * SB
            s = [None] * SB
            q = [None] * SB
            for f in range(F):
                for be in range(SB):
                    v = buf[f * CB + base + be]
                    if f == 0:
                        s[be] = v
                        q[be] = v * v
                    else:
                        s[be] = s[be] + v
                        q[be] = q[be] + v * v
            acc = jnp.zeros((16,), jnp.float32)
            for be in range(SB):
                r = s[be] * s[be] - q[be]
                acc = jnp.where(lane == be, jnp.sum(r), acc)
            out_v[pl.ds(c * CB + base, SB)] = acc * 0.5

    start(0, buf0, sem0)
    start(1, buf1, sem1)

    @pl.loop(0, CHUNKS // 2)
    def _(g):
        c = g * 2
        wait(buf0, sem0)
        compute(buf0, c)

        @pl.when(c + 2 < CHUNKS)
        def _():
            start(c + 2, buf0, sem0)

        wait(buf1, sem1)
        compute(buf1, c + 1)

        @pl.when(c + 3 < CHUNKS)
        def _():
            start(c + 3, buf1, sem1)

    pltpu.sync_copy(out_v, out_hbm.at[pl.ds(wbase, BPW)])


@jax.jit
def kernel(indices, player_v):
    idx_t = jnp.transpose(indices.astype(jnp.int32))
    mesh = plsc.VectorSubcoreMesh(
        core_axis_name="c", subcore_axis_name="s", num_cores=NC, num_subcores=NS
    )
    fm = pl.kernel(
        _fm_body,
        out_type=jax.ShapeDtypeStruct((B,), jnp.float32),
        mesh=mesh,
        compiler_params=pltpu.CompilerParams(
            needs_layout_passes=False, use_tc_tiling_on_sc=False
        ),
        scratch_types=[
            pltpu.VMEM((F, BPW), jnp.int32),
            pltpu.VMEM((F * CB, K), jnp.float32),
            pltpu.VMEM((F * CB, K), jnp.float32),
            pltpu.VMEM((BPW,), jnp.float32),
            pltpu.SemaphoreType.DMA,
            pltpu.SemaphoreType.DMA,
            pltpu.SemaphoreType.DMA,
        ],
    )
    return fm(idx_t, player_v)
